# baseline (speedup 1.0000x reference)
import functools

import jax
import jax.numpy as jnp
from jax import lax
from jax.experimental import pallas as pl
from jax.experimental.pallas import tpu as pltpu
from jax.experimental.pallas import tpu_sc as plsc

NC = 2
NS = 16
NW = NC * NS
CHUNK = 80
TL = 640
BN_EPS = 1e-5


def _make_gather(rows_total, d, nchunks):
    mesh = plsc.VectorSubcoreMesh(core_axis_name="c", subcore_axis_name="s",
                                  num_cores=NC, num_subcores=NS)
    rows_per_w = nchunks * CHUNK

    K = 4
    ng = nchunks // K

    @functools.partial(
        pl.kernel,
        out_type=jax.ShapeDtypeStruct((rows_total, d), jnp.float32),
        mesh=mesh,
        scratch_types=[
            pltpu.VMEM((nchunks, CHUNK), jnp.int32),
            [pltpu.VMEM((CHUNK, d), jnp.float32)] * K,
            [pltpu.SemaphoreType.DMA] * K,
            pltpu.SemaphoreType.DMA,
        ],
    )
    def gather(table_hbm, idx_hbm, out_hbm, idx_v, rows, gsems, wsem):
        wid = lax.axis_index("s") * NC + lax.axis_index("c")
        pltpu.sync_copy(idx_hbm.at[wid], idx_v)
        base = wid * rows_per_w

        def dst(i):
            return out_hbm.at[pl.ds(base + i * CHUNK, CHUNK)]

        def group(g, carry):
            i0 = g * K
            gd = [pltpu.async_copy(table_hbm.at[idx_v.at[i0 + j]], rows[j],
                                   gsems[j]) for j in range(K)]
            wd = []
            for j in range(K):
                gd[j].wait()
                wd.append(pltpu.async_copy(rows[j], dst(i0 + j), wsem))
            for j in range(K):
                wd[j].wait()
            return carry

        lax.fori_loop(0, ng, group, 0)
        for i in range(ng * K, nchunks):
            pltpu.async_copy(table_hbm.at[idx_v.at[i]], rows[0], gsems[0]).wait()
            pltpu.sync_copy(rows[0], dst(i))

    return gather


def _make_scatter(bsz, l, d, n_pad, nchunks):
    mesh = plsc.VectorSubcoreMesh(core_axis_name="c", subcore_axis_name="s",
                                  num_cores=NC, num_subcores=NS)
    rows_per_w = nchunks * CHUNK
    seg = n_pad // NS
    wch = CHUNK
    nw_seg = seg // wch
    K = 3
    ng = nchunks // K

    @functools.partial(
        pl.kernel,
        out_type=jax.ShapeDtypeStruct((bsz, n_pad, d), jnp.float32),
        mesh=mesh,
        scratch_types=[
            pltpu.VMEM((nchunks, CHUNK), jnp.int32),
            [pltpu.VMEM((CHUNK, d), jnp.float32)] * K,
            [pltpu.SemaphoreType.DMA] * K,
            pltpu.SemaphoreType.DMA,
            pltpu.VMEM_SHARED((n_pad, d), jnp.float32),
        ],
    )
    def scatter(lup_hbm, idx_hbm, zeros_hbm, sums_hbm,
                idx_v, rows, lsems, ssem, sums_acc):
        stage_v = rows[0]
        c = lax.axis_index("c")
        s = lax.axis_index("s")
        pltpu.sync_copy(idx_hbm.at[c, s], idx_v)

        def zinit(k, carry):
            off = s * seg + k * wch
            pltpu.sync_copy(zeros_hbm.at[pl.ds(k * wch, wch)], stage_v)
            pltpu.sync_copy(stage_v, sums_acc.at[pl.ds(off, wch)])
            return carry

        lax.fori_loop(0, nw_seg, zinit, 0)
        plsc.subcore_barrier()
        base = s * rows_per_w

        def src_at(i):
            return lup_hbm.at[c, pl.ds(base + i * CHUNK, CHUNK)]

        def group(g, carry):
            i0 = g * K
            ld = [pltpu.async_copy(src_at(i0 + j), rows[j], lsems[j])
                  for j in range(K)]
            sd = []
            for j in range(K):
                ld[j].wait()
                sd.append(pltpu.async_copy(rows[j], sums_acc.at[idx_v.at[i0 + j]],
                                           ssem, add=True))
            for j in range(K):
                sd[j].wait()
            return carry

        lax.fori_loop(0, ng, group, 0)
        for i in range(ng * K, nchunks):
            pltpu.sync_copy(src_at(i), rows[0])
            pltpu.sync_copy(rows[0], sums_acc.at[idx_v.at[i]], add=True)
        plsc.subcore_barrier()

        def wback(k, carry):
            off = s * seg + k * wch
            pltpu.sync_copy(sums_acc.at[pl.ds(off, wch)], stage_v)
            pltpu.sync_copy(stage_v, sums_hbm.at[c, pl.ds(off, wch)])
            return carry

        lax.fori_loop(0, nw_seg, wback, 0)

    return scatter


def _make_counts(bsz, l, d, n_pad, nchunks):
    mesh = plsc.VectorSubcoreMesh(core_axis_name="c", subcore_axis_name="s",
                                  num_cores=NC, num_subcores=NS)
    seg = n_pad // NS
    wch = 64
    nw_seg = seg // wch
    K = 8
    ng = nchunks // K

    @functools.partial(
        pl.kernel,
        out_type=jax.ShapeDtypeStruct((bsz, n_pad, d), jnp.float32),
        mesh=mesh,
        scratch_types=[
            pltpu.VMEM((nchunks, CHUNK), jnp.int32),
            pltpu.VMEM((CHUNK, d), jnp.float32),
            pltpu.VMEM((wch, d), jnp.float32),
            pltpu.SemaphoreType.DMA,
            pltpu.VMEM_SHARED((n_pad, d), jnp.float32),
        ],
    )
    def counts(idx_hbm, zeros_hbm, ones_hbm, cnt_hbm,
               idx_v, ones_v, stage_v, ssem, cnt_acc):
        c = lax.axis_index("c")
        s = lax.axis_index("s")
        pltpu.sync_copy(idx_hbm.at[c, s], idx_v)
        pltpu.sync_copy(ones_hbm, ones_v)

        def zinit(k, carry):
            off = s * seg + k * wch
            pltpu.sync_copy(zeros_hbm.at[pl.ds(k * wch, wch)], stage_v)
            pltpu.sync_copy(stage_v, cnt_acc.at[pl.ds(off, wch)])
            return carry

        lax.fori_loop(0, nw_seg, zinit, 0)
        plsc.subcore_barrier()

        def group(g, carry):
            i0 = g * K
            sd = [pltpu.async_copy(ones_v, cnt_acc.at[idx_v.at[i0 + j]],
                                   ssem, add=True) for j in range(K)]
            for j in range(K):
                sd[j].wait()
            return carry

        lax.fori_loop(0, ng, group, 0)
        for i in range(ng * K, nchunks):
            pltpu.sync_copy(ones_v, cnt_acc.at[idx_v.at[i]], add=True)
        plsc.subcore_barrier()

        def wback(k, carry):
            off = s * seg + k * wch
            pltpu.sync_copy(cnt_acc.at[pl.ds(off, wch)], stage_v)
            pltpu.sync_copy(stage_v, cnt_hbm.at[c, pl.ds(off, wch)])
            return carry

        lax.fori_loop(0, nw_seg, wback, 0)

    return counts


def _flip_pairs(g):
    up = jnp.roll(g, -1, axis=0)
    dn = jnp.roll(g, 1, axis=0)
    rows = lax.broadcasted_iota(jnp.int32, g.shape, 0)
    return jnp.where(rows % 2 == 0, up, dn)


def _stats_body(g_ref, enc_ref, w1t_ref, b1_ref, out_ref):
    b = pl.program_id(0)
    t = pl.program_id(1)
    g = g_ref[0]
    msg = jnp.concatenate([g, _flip_pairs(g), enc_ref[0].T], axis=1)
    h = jnp.dot(msg.astype(jnp.bfloat16), w1t_ref[...],
                preferred_element_type=jnp.float32) + b1_ref[...]

    @pl.when((b == 0) & (t == 0))
    def _():
        out_ref[...] = jnp.zeros_like(out_ref)

    out_ref[0:1, :] += jnp.sum(h, axis=0, keepdims=True)
    out_ref[1:2, :] += jnp.sum(h * h, axis=0, keepdims=True)


def _mlp_body(g_ref, enc_ref, w1s_ref, b1s_ref, w2t_ref, b2_ref, lup_ref):
    g = g_ref[0]
    msg = jnp.concatenate([g, _flip_pairs(g), enc_ref[0].T], axis=1)
    h = jnp.dot(msg.astype(jnp.bfloat16), w1s_ref[...],
                preferred_element_type=jnp.float32) + b1s_ref[...]
    h = jnp.maximum(h, 0.0)
    lup_ref[0] = jnp.dot(h.astype(jnp.bfloat16), w2t_ref[...],
                         preferred_element_type=jnp.float32) + b2_ref[...]


def _combine_body(ld_ref, sums_ref, cnt_ref, out_ref):
    s_t = sums_ref[0].T
    cnt = jnp.maximum(cnt_ref[0].T[0:1, :], 1.0)
    out_ref[0] = ld_ref[0] + s_t / cnt


def _stream(ldesc, line_enc, idx, W1, b1, bn_g, bn_b, W2, b2):
    bsz, d, n = ldesc.shape
    l = idx.shape[1]
    rows_total = bsz * l
    nchunks_g = rows_total // (NW * CHUNK)
    nchunks_s = l // (NS * CHUNK)
    n_pad = ((n + NS * 128 - 1) // (NS * 128)) * (NS * 128)
    nt = l // TL

    table = jnp.swapaxes(ldesc, 1, 2).reshape(bsz * n, d)
    gidx = (idx + (jnp.arange(bsz, dtype=jnp.int32) * n)[:, None])
    gidx3 = gidx.reshape(NW, nchunks_g, CHUNK)

    gathered = _make_gather(rows_total, d, nchunks_g)(table, gidx3)
    gathered = gathered.reshape(bsz, l, d)

    w1t = W1.T
    b1r = b1.reshape(1, -1)
    grid = (bsz, nt)
    stats = pl.pallas_call(
        _stats_body,
        grid=grid,
        in_specs=[
            pl.BlockSpec((1, TL, d), lambda b, t: (b, t, 0)),
            pl.BlockSpec((1, d, TL), lambda b, t: (b, 0, t)),
            pl.BlockSpec((3 * d, 2 * d), lambda b, t: (0, 0)),
            pl.BlockSpec((1, 2 * d), lambda b, t: (0, 0)),
        ],
        out_specs=pl.BlockSpec((8, 2 * d), lambda b, t: (0, 0)),
        out_shape=jax.ShapeDtypeStruct((8, 2 * d), jnp.float32),
        compiler_params=pltpu.CompilerParams(
            dimension_semantics=("arbitrary", "arbitrary")),
    )(gathered, line_enc, w1t.astype(jnp.bfloat16), b1r)

    cnt_total = rows_total
    mu = stats[0] / cnt_total
    var = stats[1] / cnt_total - mu * mu
    scale = bn_g / jnp.sqrt(var + BN_EPS)
    w1s = w1t * scale[None, :]
    b1s = ((b1 - mu) * scale + bn_b).reshape(1, -1)

    lup = pl.pallas_call(
        _mlp_body,
        grid=grid,
        in_specs=[
            pl.BlockSpec((1, TL, d), lambda b, t: (b, t, 0)),
            pl.BlockSpec((1, d, TL), lambda b, t: (b, 0, t)),
            pl.BlockSpec((3 * d, 2 * d), lambda b, t: (0, 0)),
            pl.BlockSpec((1, 2 * d), lambda b, t: (0, 0)),
            pl.BlockSpec((2 * d, d), lambda b, t: (0, 0)),
            pl.BlockSpec((1, d), lambda b, t: (0, 0)),
        ],
        out_specs=pl.BlockSpec((1, TL, d), lambda b, t: (b, t, 0)),
        out_shape=jax.ShapeDtypeStruct((bsz, l, d), jnp.float32),
        compiler_params=pltpu.CompilerParams(
            dimension_semantics=("arbitrary", "arbitrary")),
    )(gathered, line_enc, w1s.astype(jnp.bfloat16), b1s,
      W2.T.astype(jnp.bfloat16), b2.reshape(1, -1))

    idx4 = idx.reshape(bsz, NS, nchunks_s, CHUNK)
    z128 = jnp.zeros((n_pad, d), jnp.float32)
    ones = jnp.ones((CHUNK, d), jnp.float32)
    sums = _make_scatter(bsz, l, d, n_pad, nchunks_s)(lup, idx4, z128)
    cnts = _make_counts(bsz, l, d, n_pad, nchunks_s)(idx4, z128, ones)

    out = pl.pallas_call(
        _combine_body,
        grid=(bsz, (n + 127) // 128),
        in_specs=[
            pl.BlockSpec((1, d, 128), lambda b, t: (b, 0, t)),
            pl.BlockSpec((1, 128, d), lambda b, t: (b, t, 0)),
            pl.BlockSpec((1, 128, d), lambda b, t: (b, t, 0)),
        ],
        out_specs=pl.BlockSpec((1, d, 128), lambda b, t: (b, 0, t)),
        out_shape=jax.ShapeDtypeStruct((bsz, d, n), jnp.float32),
        compiler_params=pltpu.CompilerParams(
            dimension_semantics=("arbitrary", "arbitrary")),
    )(ldesc, sums, cnts)
    return out


def kernel(ldesc0, ldesc1, line_enc0, line_enc1, lines_junc_idx0,
           lines_junc_idx1, W1, b1, bn_g, bn_b, W2, b2):
    out0 = _stream(ldesc0, line_enc0, lines_junc_idx0,
                   W1, b1, bn_g, bn_b, W2, b2)
    out1 = _stream(ldesc1, line_enc1, lines_junc_idx1,
                   W1, b1, bn_g, bn_b, W2, b2)
    return (out0, out1)


# speedup vs baseline: 1.0031x; 1.0031x over previous
import functools

import jax
import jax.numpy as jnp
from jax import lax
from jax.experimental import pallas as pl
from jax.experimental.pallas import tpu as pltpu
from jax.experimental.pallas import tpu_sc as plsc

NC = 2
NS = 16
NW = NC * NS
CHUNK = 80
TL = 640
BN_EPS = 1e-5


def _make_gather(rows_total, d, nchunks):
    mesh = plsc.VectorSubcoreMesh(core_axis_name="c", subcore_axis_name="s",
                                  num_cores=NC, num_subcores=NS)
    rows_per_w = nchunks * CHUNK

    K = 4
    ng = nchunks // K

    @functools.partial(
        pl.kernel,
        out_type=jax.ShapeDtypeStruct((rows_total, d), jnp.float32),
        mesh=mesh,
        scratch_types=[
            pltpu.VMEM((nchunks, CHUNK), jnp.int32),
            [pltpu.VMEM((CHUNK, d), jnp.float32)] * K,
            [pltpu.SemaphoreType.DMA] * K,
            pltpu.SemaphoreType.DMA,
        ],
    )
    def gather(table_hbm, idx_hbm, out_hbm, idx_v, rows, gsems, wsem):
        wid = lax.axis_index("s") * NC + lax.axis_index("c")
        pltpu.sync_copy(idx_hbm.at[wid], idx_v)
        base = wid * rows_per_w

        def dst(i):
            return out_hbm.at[pl.ds(base + i * CHUNK, CHUNK)]

        def group(g, carry):
            i0 = g * K
            gd = [pltpu.async_copy(table_hbm.at[idx_v.at[i0 + j]], rows[j],
                                   gsems[j]) for j in range(K)]
            wd = []
            for j in range(K):
                gd[j].wait()
                wd.append(pltpu.async_copy(rows[j], dst(i0 + j), wsem))
            for j in range(K):
                wd[j].wait()
            return carry

        lax.fori_loop(0, ng, group, 0)
        for i in range(ng * K, nchunks):
            pltpu.async_copy(table_hbm.at[idx_v.at[i]], rows[0], gsems[0]).wait()
            pltpu.sync_copy(rows[0], dst(i))

    return gather


def _make_scatter(bsz, l, d, n_pad, nchunks):
    mesh = plsc.VectorSubcoreMesh(core_axis_name="c", subcore_axis_name="s",
                                  num_cores=NC, num_subcores=NS)
    rows_per_w = nchunks * CHUNK
    seg = n_pad // NS
    wch = CHUNK
    nw_seg = seg // wch
    K = 3
    ng = nchunks // K

    @functools.partial(
        pl.kernel,
        out_type=jax.ShapeDtypeStruct((bsz, n_pad, d), jnp.float32),
        mesh=mesh,
        scratch_types=[
            pltpu.VMEM((nchunks, CHUNK), jnp.int32),
            [pltpu.VMEM((CHUNK, d), jnp.float32)] * K,
            [pltpu.SemaphoreType.DMA] * K,
            pltpu.SemaphoreType.DMA,
            pltpu.VMEM_SHARED((n_pad, d), jnp.float32),
        ],
    )
    def scatter(lup_hbm, idx_hbm, zeros_hbm, sums_hbm,
                idx_v, rows, lsems, ssem, sums_acc):
        stage_v = rows[0]
        c = lax.axis_index("c")
        s = lax.axis_index("s")
        pltpu.sync_copy(idx_hbm.at[c, s], idx_v)

        def zinit(k, carry):
            off = s * seg + k * wch
            pltpu.sync_copy(zeros_hbm.at[pl.ds(k * wch, wch)], stage_v)
            pltpu.sync_copy(stage_v, sums_acc.at[pl.ds(off, wch)])
            return carry

        lax.fori_loop(0, nw_seg, zinit, 0)
        plsc.subcore_barrier()
        base = s * rows_per_w

        def src_at(i):
            return lup_hbm.at[c, pl.ds(base + i * CHUNK, CHUNK)]

        def group(g, carry):
            i0 = g * K
            ld = [pltpu.async_copy(src_at(i0 + j), rows[j], lsems[j])
                  for j in range(K)]
            sd = []
            for j in range(K):
                ld[j].wait()
                sd.append(pltpu.async_copy(rows[j], sums_acc.at[idx_v.at[i0 + j]],
                                           ssem, add=True))
            for j in range(K):
                sd[j].wait()
            return carry

        lax.fori_loop(0, ng, group, 0)
        for i in range(ng * K, nchunks):
            pltpu.sync_copy(src_at(i), rows[0])
            pltpu.sync_copy(rows[0], sums_acc.at[idx_v.at[i]], add=True)
        plsc.subcore_barrier()

        def wback(k, carry):
            off = s * seg + k * wch
            pltpu.sync_copy(sums_acc.at[pl.ds(off, wch)], stage_v)
            pltpu.sync_copy(stage_v, sums_hbm.at[c, pl.ds(off, wch)])
            return carry

        lax.fori_loop(0, nw_seg, wback, 0)

    return scatter


def _make_counts(bsz, l, d, n_pad, nchunks):
    mesh = plsc.VectorSubcoreMesh(core_axis_name="c", subcore_axis_name="s",
                                  num_cores=NC, num_subcores=NS)
    seg = n_pad // NS
    wch = 64
    nw_seg = seg // wch
    K = 8
    ng = nchunks // K

    @functools.partial(
        pl.kernel,
        out_type=jax.ShapeDtypeStruct((bsz, n_pad, d), jnp.float32),
        mesh=mesh,
        scratch_types=[
            pltpu.VMEM((nchunks, CHUNK), jnp.int32),
            pltpu.VMEM((CHUNK, d), jnp.float32),
            pltpu.VMEM((wch, d), jnp.float32),
            pltpu.SemaphoreType.DMA,
            pltpu.VMEM_SHARED((n_pad, d), jnp.float32),
        ],
    )
    def counts(idx_hbm, zeros_hbm, ones_hbm, cnt_hbm,
               idx_v, ones_v, stage_v, ssem, cnt_acc):
        c = lax.axis_index("c")
        s = lax.axis_index("s")
        pltpu.sync_copy(idx_hbm.at[c, s], idx_v)
        pltpu.sync_copy(ones_hbm, ones_v)

        def zinit(k, carry):
            off = s * seg + k * wch
            pltpu.sync_copy(zeros_hbm.at[pl.ds(k * wch, wch)], stage_v)
            pltpu.sync_copy(stage_v, cnt_acc.at[pl.ds(off, wch)])
            return carry

        lax.fori_loop(0, nw_seg, zinit, 0)
        plsc.subcore_barrier()

        def group(g, carry):
            i0 = g * K
            sd = [pltpu.async_copy(ones_v, cnt_acc.at[idx_v.at[i0 + j]],
                                   ssem, add=True) for j in range(K)]
            for j in range(K):
                sd[j].wait()
            return carry

        lax.fori_loop(0, ng, group, 0)
        for i in range(ng * K, nchunks):
            pltpu.sync_copy(ones_v, cnt_acc.at[idx_v.at[i]], add=True)
        plsc.subcore_barrier()

        def wback(k, carry):
            off = s * seg + k * wch
            pltpu.sync_copy(cnt_acc.at[pl.ds(off, wch)], stage_v)
            pltpu.sync_copy(stage_v, cnt_hbm.at[c, pl.ds(off, wch)])
            return carry

        lax.fori_loop(0, nw_seg, wback, 0)

    return counts


def _flip_pairs(g):
    up = jnp.roll(g, -1, axis=0)
    dn = jnp.roll(g, 1, axis=0)
    rows = lax.broadcasted_iota(jnp.int32, g.shape, 0)
    return jnp.where(rows % 2 == 0, up, dn)


def _stats_body(g_ref, enc_ref, w1t_ref, b1_ref, out_ref):
    b = pl.program_id(0)
    t = pl.program_id(1)
    g = g_ref[0]
    msg = jnp.concatenate([g, _flip_pairs(g), enc_ref[0].T], axis=1)
    h = jnp.dot(msg, w1t_ref[...], preferred_element_type=jnp.float32) + b1_ref[...]

    @pl.when((b == 0) & (t == 0))
    def _():
        out_ref[...] = jnp.zeros_like(out_ref)

    out_ref[0:1, :] += jnp.sum(h, axis=0, keepdims=True)
    out_ref[1:2, :] += jnp.sum(h * h, axis=0, keepdims=True)


def _mlp_body(g_ref, enc_ref, w1s_ref, b1s_ref, w2t_ref, b2_ref, lup_ref):
    g = g_ref[0]
    msg = jnp.concatenate([g, _flip_pairs(g), enc_ref[0].T], axis=1)
    h = jnp.dot(msg, w1s_ref[...], preferred_element_type=jnp.float32) + b1s_ref[...]
    h = jnp.maximum(h, 0.0)
    lup_ref[0] = jnp.dot(h, w2t_ref[...], preferred_element_type=jnp.float32) + b2_ref[...]


def _combine_body(ld_ref, sums_ref, cnt_ref, out_ref):
    s_t = sums_ref[0].T
    cnt = jnp.maximum(cnt_ref[0].T[0:1, :], 1.0)
    out_ref[0] = ld_ref[0] + s_t / cnt


def _stream(ldesc, line_enc, idx, W1, b1, bn_g, bn_b, W2, b2):
    bsz, d, n = ldesc.shape
    l = idx.shape[1]
    rows_total = bsz * l
    nchunks_g = rows_total // (NW * CHUNK)
    nchunks_s = l // (NS * CHUNK)
    n_pad = ((n + NS * 128 - 1) // (NS * 128)) * (NS * 128)
    nt = l // TL

    table = jnp.swapaxes(ldesc, 1, 2).reshape(bsz * n, d)
    gidx = (idx + (jnp.arange(bsz, dtype=jnp.int32) * n)[:, None])
    gidx3 = gidx.reshape(NW, nchunks_g, CHUNK)

    gathered = _make_gather(rows_total, d, nchunks_g)(table, gidx3)
    gathered = gathered.reshape(bsz, l, d)

    w1t = W1.T
    b1r = b1.reshape(1, -1)
    grid = (bsz, nt)
    stats = pl.pallas_call(
        _stats_body,
        grid=grid,
        in_specs=[
            pl.BlockSpec((1, TL, d), lambda b, t: (b, t, 0)),
            pl.BlockSpec((1, d, TL), lambda b, t: (b, 0, t)),
            pl.BlockSpec((3 * d, 2 * d), lambda b, t: (0, 0)),
            pl.BlockSpec((1, 2 * d), lambda b, t: (0, 0)),
        ],
        out_specs=pl.BlockSpec((8, 2 * d), lambda b, t: (0, 0)),
        out_shape=jax.ShapeDtypeStruct((8, 2 * d), jnp.float32),
        compiler_params=pltpu.CompilerParams(
            dimension_semantics=("arbitrary", "arbitrary")),
    )(gathered, line_enc, w1t, b1r)

    cnt_total = rows_total
    mu = stats[0] / cnt_total
    var = stats[1] / cnt_total - mu * mu
    scale = bn_g / jnp.sqrt(var + BN_EPS)
    w1s = w1t * scale[None, :]
    b1s = ((b1 - mu) * scale + bn_b).reshape(1, -1)

    lup = pl.pallas_call(
        _mlp_body,
        grid=grid,
        in_specs=[
            pl.BlockSpec((1, TL, d), lambda b, t: (b, t, 0)),
            pl.BlockSpec((1, d, TL), lambda b, t: (b, 0, t)),
            pl.BlockSpec((3 * d, 2 * d), lambda b, t: (0, 0)),
            pl.BlockSpec((1, 2 * d), lambda b, t: (0, 0)),
            pl.BlockSpec((2 * d, d), lambda b, t: (0, 0)),
            pl.BlockSpec((1, d), lambda b, t: (0, 0)),
        ],
        out_specs=pl.BlockSpec((1, TL, d), lambda b, t: (b, t, 0)),
        out_shape=jax.ShapeDtypeStruct((bsz, l, d), jnp.float32),
        compiler_params=pltpu.CompilerParams(
            dimension_semantics=("arbitrary", "arbitrary")),
    )(gathered, line_enc, w1s, b1s, W2.T, b2.reshape(1, -1))

    idx4 = idx.reshape(bsz, NS, nchunks_s, CHUNK)
    z128 = jnp.zeros((n_pad, d), jnp.float32)
    ones = jnp.ones((CHUNK, d), jnp.float32)
    sums = _make_scatter(bsz, l, d, n_pad, nchunks_s)(lup, idx4, z128)
    cnts = _make_counts(bsz, l, d, n_pad, nchunks_s)(idx4, z128, ones)

    out = pl.pallas_call(
        _combine_body,
        grid=(bsz, (n + 127) // 128),
        in_specs=[
            pl.BlockSpec((1, d, 128), lambda b, t: (b, 0, t)),
            pl.BlockSpec((1, 128, d), lambda b, t: (b, t, 0)),
            pl.BlockSpec((1, 128, d), lambda b, t: (b, t, 0)),
        ],
        out_specs=pl.BlockSpec((1, d, 128), lambda b, t: (b, 0, t)),
        out_shape=jax.ShapeDtypeStruct((bsz, d, n), jnp.float32),
        compiler_params=pltpu.CompilerParams(
            dimension_semantics=("arbitrary", "arbitrary")),
    )(ldesc, sums, cnts)
    return out


def kernel(ldesc0, ldesc1, line_enc0, line_enc1, lines_junc_idx0,
           lines_junc_idx1, W1, b1, bn_g, bn_b, W2, b2):
    out0 = _stream(ldesc0, line_enc0, lines_junc_idx0,
                   W1, b1, bn_g, bn_b, W2, b2)
    out1 = _stream(ldesc1, line_enc1, lines_junc_idx1,
                   W1, b1, bn_g, bn_b, W2, b2)
    return (out0, out1)
